# Initial kernel scaffold; baseline (speedup 1.0000x reference)
import functools

import jax
import jax.numpy as jnp
from jax import lax
from jax.experimental import pallas as pl
from jax.experimental.pallas import tpu as pltpu
from jax.experimental.pallas import tpu_sc as plsc

_LANES = 16
_NUM_TILES = 16
_CHUNK = 128


def _round_up(v, m):
    return (v + m - 1) // m * m


def _matmul_halves(x, w):
    n, d_in = x.shape
    d_out = w.shape[1]
    h = d_out // 2
    blk = 2000
    grid = n // blk

    def body(x_ref, w_ref, out0_ref, out1_ref):
        s = jnp.dot(x_ref[...], w_ref[...], preferred_element_type=jnp.float32)
        out0_ref[...] = s[:, :h]
        out1_ref[...] = s[:, h:]

    return pl.pallas_call(
        body,
        grid=(grid,),
        in_specs=[
            pl.BlockSpec((blk, d_in), lambda i: (i, 0)),
            pl.BlockSpec((d_in, d_out), lambda i: (0, 0)),
        ],
        out_specs=[
            pl.BlockSpec((blk, h), lambda i: (i, 0)),
            pl.BlockSpec((blk, h), lambda i: (i, 0)),
        ],
        out_shape=[
            jax.ShapeDtypeStruct((n, h), jnp.float32),
            jax.ShapeDtypeStruct((n, h), jnp.float32),
        ],
    )(x, w)


def _make_sc_scatter(n_pad, h, ept):
    rows_per_tile = n_pad // _NUM_TILES
    chunks = ept // _CHUNK
    mesh = plsc.VectorSubcoreMesh(core_axis_name="c", subcore_axis_name="s")

    @functools.partial(
        pl.kernel,
        out_type=jax.ShapeDtypeStruct((n_pad, 2 * h), jnp.float32),
        mesh=mesh,
        scratch_types=[
            pltpu.VMEM((_CHUNK,), jnp.int32),
            pltpu.VMEM((_CHUNK,), jnp.int32),
            pltpu.VMEM((_CHUNK,), jnp.float32),
            pltpu.VMEM((_CHUNK, h), jnp.float32),
            pltpu.VMEM_SHARED((n_pad, h), jnp.float32),
            pltpu.SemaphoreType.DMA,
        ],
    )
    def sc_kernel(sup0, sup1, rowh, colh, ewh, biash, out_hbm,
                  rowv, colv, wv, gbuf, acc, gsem):
        c = lax.axis_index("c")
        s = lax.axis_index("s")
        r0 = s * rows_per_tile

        def run(sup, coff):
            pltpu.sync_copy(
                biash.at[pl.ds(r0, rows_per_tile), pl.ds(coff, h)],
                acc.at[pl.ds(r0, rows_per_tile)],
            )
            plsc.subcore_barrier()

            @pl.loop(0, chunks)
            def _chunk(k):
                base = s * ept + k * _CHUNK
                pltpu.sync_copy(rowh.at[pl.ds(base, _CHUNK)], rowv)
                pltpu.sync_copy(colh.at[pl.ds(base, _CHUNK)], colv)
                pltpu.sync_copy(ewh.at[pl.ds(base, _CHUNK)], wv)
                pltpu.async_copy(sup.at[rowv], gbuf, gsem).wait()

                @pl.loop(0, _CHUNK // _LANES)
                def _grp(g):
                    wvec = wv[pl.ds(g * _LANES, _LANES)]
                    for t in range(_LANES):
                        spl = jnp.take(
                            wvec,
                            jnp.full((_LANES,), t, jnp.int32),
                            mode="promise_in_bounds",
                        )
                        e = g * _LANES + t
                        for j in range(h // _LANES):
                            sl = pl.ds(j * _LANES, _LANES)
                            gbuf[e, sl] = gbuf[e, sl] * spl

                pltpu.sync_copy(gbuf, acc.at[colv], add=True)

            plsc.subcore_barrier()
            pltpu.sync_copy(
                acc.at[pl.ds(r0, rows_per_tile)],
                out_hbm.at[pl.ds(r0, rows_per_tile), pl.ds(coff, h)],
            )

        @pl.when(c == 0)
        def _():
            run(sup0, 0)

        @pl.when(c == 1)
        def _():
            run(sup1, h)

    return sc_kernel


def kernel(x, edge_index, edge_weight, W, b):
    n, _ = x.shape
    d_out = W.shape[1]
    h = d_out // 2
    e = edge_weight.shape[0]

    sup0, sup1 = _matmul_halves(x, W)

    n_pad = _round_up(n, _NUM_TILES * 8)
    ept = _round_up(-(-e // _NUM_TILES), _CHUNK)
    e_pad = ept * _NUM_TILES
    pad = e_pad - e

    row = edge_index[0]
    col = edge_index[1]
    rowp = jnp.concatenate([row, jnp.zeros((pad,), jnp.int32)])
    colp = jnp.concatenate([col, jnp.zeros((pad,), jnp.int32)])
    ewp = jnp.concatenate([edge_weight, jnp.zeros((pad,), jnp.float32)])
    bias_full = jnp.broadcast_to(b, (n_pad, d_out))

    sc = _make_sc_scatter(n_pad, h, ept)
    out = sc(sup0, sup1, rowp, colp, ewp, bias_full)
    return out[:n]


# speedup vs baseline: 2.9193x; 2.9193x over previous
import functools

import jax
import jax.numpy as jnp
from jax import lax
from jax.experimental import pallas as pl
from jax.experimental.pallas import tpu as pltpu
from jax.experimental.pallas import tpu_sc as plsc

_LANES = 16
_NUM_TILES = 16
_CHUNK = 128


def _round_up(v, m):
    return (v + m - 1) // m * m


def _matmul_halves(x, w):
    n, d_in = x.shape
    d_out = w.shape[1]
    h = d_out // 2
    blk = 2000
    grid = n // blk

    def body(x_ref, w_ref, out0_ref, out1_ref):
        s = jnp.dot(x_ref[...], w_ref[...], preferred_element_type=jnp.float32)
        out0_ref[...] = s[:, :h]
        out1_ref[...] = s[:, h:]

    return pl.pallas_call(
        body,
        grid=(grid,),
        in_specs=[
            pl.BlockSpec((blk, d_in), lambda i: (i, 0)),
            pl.BlockSpec((d_in, d_out), lambda i: (0, 0)),
        ],
        out_specs=[
            pl.BlockSpec((blk, h), lambda i: (i, 0)),
            pl.BlockSpec((blk, h), lambda i: (i, 0)),
        ],
        out_shape=[
            jax.ShapeDtypeStruct((n, h), jnp.float32),
            jax.ShapeDtypeStruct((n, h), jnp.float32),
        ],
    )(x, w)


def _make_sc_scatter(n_pad, h, ept):
    rows_per_tile = n_pad // _NUM_TILES
    chunks = ept // _CHUNK
    mesh = plsc.VectorSubcoreMesh(core_axis_name="c", subcore_axis_name="s")

    @functools.partial(
        pl.kernel,
        out_type=jax.ShapeDtypeStruct((n_pad, 2 * h), jnp.float32),
        mesh=mesh,
        scratch_types=[
            pltpu.VMEM((_CHUNK,), jnp.int32),
            pltpu.VMEM((_CHUNK,), jnp.int32),
            pltpu.VMEM((_CHUNK,), jnp.float32),
            pltpu.VMEM((_CHUNK, h), jnp.float32),
            pltpu.VMEM_SHARED((n_pad, h), jnp.float32),
            pltpu.SemaphoreType.DMA,
        ],
    )
    def sc_kernel(sup0, sup1, rowh, colh, ewh, biash, out_hbm,
                  rowv, colv, wv, gbuf, acc, gsem):
        c = lax.axis_index("c")
        s = lax.axis_index("s")
        r0 = s * rows_per_tile

        def run(sup, coff):
            pltpu.sync_copy(
                biash.at[pl.ds(r0, rows_per_tile), pl.ds(coff, h)],
                acc.at[pl.ds(r0, rows_per_tile)],
            )
            plsc.subcore_barrier()

            @pl.loop(0, chunks)
            def _chunk(k):
                base = s * ept + k * _CHUNK
                pltpu.sync_copy(rowh.at[pl.ds(base, _CHUNK)], rowv)
                pltpu.sync_copy(colh.at[pl.ds(base, _CHUNK)], colv)
                pltpu.sync_copy(ewh.at[pl.ds(base, _CHUNK)], wv)
                pltpu.async_copy(sup.at[rowv], gbuf, gsem).wait()

                dnums = lax.GatherDimensionNumbers(
                    offset_dims=(), collapsed_slice_dims=(0,),
                    start_index_map=(0,))

                @pl.loop(0, _CHUNK // _LANES)
                def _grp(g):
                    wvec = wv[pl.ds(g * _LANES, _LANES)]
                    for t in range(_LANES):
                        spl = lax.gather(
                            wvec,
                            jnp.full((_LANES, 1), t, jnp.int32),
                            dnums, (1,),
                            mode=lax.GatherScatterMode.PROMISE_IN_BOUNDS,
                        )
                        e = g * _LANES + t
                        for j in range(h // _LANES):
                            sl = pl.ds(j * _LANES, _LANES)
                            gbuf[e, sl] = gbuf[e, sl] * spl

                pltpu.sync_copy(gbuf, acc.at[colv], add=True)

            plsc.subcore_barrier()
            pltpu.sync_copy(
                acc.at[pl.ds(r0, rows_per_tile)],
                out_hbm.at[pl.ds(r0, rows_per_tile), pl.ds(coff, h)],
            )

        @pl.when(c == 0)
        def _():
            run(sup0, 0)

        @pl.when(c == 1)
        def _():
            run(sup1, h)

    return sc_kernel


def kernel(x, edge_index, edge_weight, W, b):
    n, _ = x.shape
    d_out = W.shape[1]
    h = d_out // 2
    e = edge_weight.shape[0]

    sup0, sup1 = _matmul_halves(x, W)

    n_pad = _round_up(n, _NUM_TILES * 8)
    ept = _round_up(-(-e // _NUM_TILES), _CHUNK)
    e_pad = ept * _NUM_TILES
    pad = e_pad - e

    row = edge_index[0]
    col = edge_index[1]
    rowp = jnp.concatenate([row, jnp.zeros((pad,), jnp.int32)])
    colp = jnp.concatenate([col, jnp.zeros((pad,), jnp.int32)])
    ewp = jnp.concatenate([edge_weight, jnp.zeros((pad,), jnp.float32)])
    bias_full = jnp.broadcast_to(b, (n_pad, d_out))

    sc = _make_sc_scatter(n_pad, h, ept)
    out = sc(sup0, sup1, rowp, colp, ewp, bias_full)
    return out[:n]


# speedup vs baseline: 3.5271x; 1.2082x over previous
import functools

import jax
import jax.numpy as jnp
from jax import lax
from jax.experimental import pallas as pl
from jax.experimental.pallas import tpu as pltpu
from jax.experimental.pallas import tpu_sc as plsc

_LANES = 16
_NUM_TILES = 16
_CHUNK = 128


def _round_up(v, m):
    return (v + m - 1) // m * m


def _matmul_halves(x, w):
    n, d_in = x.shape
    d_out = w.shape[1]
    h = d_out // 2
    blk = 2000
    grid = n // blk

    def body(x_ref, w_ref, out0_ref, out1_ref):
        s = jnp.dot(x_ref[...], w_ref[...], preferred_element_type=jnp.float32)
        out0_ref[...] = s[:, :h]
        out1_ref[...] = s[:, h:]

    return pl.pallas_call(
        body,
        grid=(grid,),
        in_specs=[
            pl.BlockSpec((blk, d_in), lambda i: (i, 0)),
            pl.BlockSpec((d_in, d_out), lambda i: (0, 0)),
        ],
        out_specs=[
            pl.BlockSpec((blk, h), lambda i: (i, 0)),
            pl.BlockSpec((blk, h), lambda i: (i, 0)),
        ],
        out_shape=[
            jax.ShapeDtypeStruct((n, h), jnp.float32),
            jax.ShapeDtypeStruct((n, h), jnp.float32),
        ],
    )(x, w)


def _make_sc_scatter(n_pad, h, ept):
    rows_per_tile = n_pad // _NUM_TILES
    chunks = ept // _CHUNK
    assert chunks % 2 == 0
    mesh = plsc.VectorSubcoreMesh(core_axis_name="c", subcore_axis_name="s")

    @functools.partial(
        pl.kernel,
        out_type=jax.ShapeDtypeStruct((n_pad, 2 * h), jnp.float32),
        mesh=mesh,
        scratch_types=[
            pltpu.VMEM((chunks, _CHUNK), jnp.int32),
            pltpu.VMEM((_CHUNK,), jnp.int32),
            pltpu.VMEM((_CHUNK,), jnp.int32),
            pltpu.VMEM((_CHUNK,), jnp.float32),
            pltpu.VMEM((_CHUNK,), jnp.float32),
            pltpu.VMEM((_CHUNK, h), jnp.float32),
            pltpu.VMEM((_CHUNK, h), jnp.float32),
            pltpu.VMEM_SHARED((n_pad, h), jnp.float32),
            pltpu.SemaphoreType.DMA,
            pltpu.SemaphoreType.DMA,
            pltpu.SemaphoreType.DMA,
            pltpu.SemaphoreType.DMA,
        ],
    )
    def sc_kernel(sup0, sup1, rowh, colh, ewh, biash, out_hbm,
                  rowv, c0, c1, w0, w1, gbuf0, gbuf1, acc,
                  gsem, ssem, csem0, csem1):
        c = lax.axis_index("c")
        s = lax.axis_index("s")
        r0 = s * rows_per_tile
        dnums = lax.GatherDimensionNumbers(
            offset_dims=(), collapsed_slice_dims=(0,), start_index_map=(0,))

        def run(sup, coff):
            def gstart(k, buf):
                pltpu.async_copy(sup.at[rowv.at[k]], buf, gsem)

            def gwait(k, buf):
                pltpu.make_async_copy(sup.at[rowv.at[k]], buf, gsem).wait()

            def sstart(buf, cbuf):
                pltpu.async_copy(buf, acc.at[cbuf], ssem, add=True)

            def swait(buf, cbuf):
                pltpu.make_async_copy(buf, acc.at[cbuf], ssem).wait()

            def cwstart(k, cbuf, wbuf, sem):
                pltpu.async_copy(colh.at[s, k, :], cbuf, sem)
                pltpu.async_copy(ewh.at[s, k, :], wbuf, sem)

            def cwwait(k, cbuf, wbuf, sem):
                pltpu.make_async_copy(colh.at[s, k, :], cbuf, sem).wait()
                pltpu.make_async_copy(ewh.at[s, k, :], wbuf, sem).wait()

            def scale(buf, wbuf):
                @pl.loop(0, _CHUNK // _LANES)
                def _grp(g):
                    wvec = wbuf[pl.ds(g * _LANES, _LANES)]
                    for t in range(_LANES):
                        spl = lax.gather(
                            wvec,
                            jnp.full((_LANES, 1), t, jnp.int32),
                            dnums, (1,),
                            mode=lax.GatherScatterMode.PROMISE_IN_BOUNDS,
                        )
                        e = g * _LANES + t
                        for j in range(h // _LANES):
                            sl = pl.ds(j * _LANES, _LANES)
                            buf[e, sl] = buf[e, sl] * spl

            dr = pltpu.async_copy(rowh.at[s, :, :], rowv, gsem)
            db = pltpu.async_copy(
                biash.at[pl.ds(r0, rows_per_tile), pl.ds(coff, h)],
                acc.at[pl.ds(r0, rows_per_tile)], ssem)
            dr.wait()
            cwstart(0, c0, w0, csem0)
            gstart(0, gbuf0)
            cwstart(1, c1, w1, csem1)
            db.wait()
            plsc.subcore_barrier()

            @pl.loop(0, chunks, step=2)
            def _pair(k):
                gwait(k, gbuf0)

                @pl.when(k > 0)
                def _():
                    swait(gbuf1, c1)
                gstart(k + 1, gbuf1)

                @pl.when(k > 0)
                def _():
                    cwstart(k + 1, c1, w1, csem1)
                cwwait(k, c0, w0, csem0)
                scale(gbuf0, w0)
                sstart(gbuf0, c0)

                gwait(k + 1, gbuf1)
                swait(gbuf0, c0)

                @pl.when(k + 2 < chunks)
                def _():
                    gstart(k + 2, gbuf0)
                    cwstart(k + 2, c0, w0, csem0)
                cwwait(k + 1, c1, w1, csem1)
                scale(gbuf1, w1)
                sstart(gbuf1, c1)

            swait(gbuf1, c1)
            plsc.subcore_barrier()
            pltpu.sync_copy(
                acc.at[pl.ds(r0, rows_per_tile)],
                out_hbm.at[pl.ds(r0, rows_per_tile), pl.ds(coff, h)],
            )

        @pl.when(c == 0)
        def _():
            run(sup0, 0)

        @pl.when(c == 1)
        def _():
            run(sup1, h)

    return sc_kernel


def kernel(x, edge_index, edge_weight, W, b):
    n, _ = x.shape
    d_out = W.shape[1]
    h = d_out // 2
    e = edge_weight.shape[0]

    sup0, sup1 = _matmul_halves(x, W)

    n_pad = _round_up(n, _NUM_TILES * 8)
    ept = _round_up(-(-e // _NUM_TILES), 2 * _CHUNK)
    e_pad = ept * _NUM_TILES
    pad = e_pad - e

    row = edge_index[0]
    col = edge_index[1]
    shp = (_NUM_TILES, ept // _CHUNK, _CHUNK)
    rowp = jnp.concatenate([row, jnp.zeros((pad,), jnp.int32)]).reshape(shp)
    colp = jnp.concatenate([col, jnp.zeros((pad,), jnp.int32)]).reshape(shp)
    ewp = jnp.concatenate(
        [edge_weight, jnp.zeros((pad,), jnp.float32)]).reshape(shp)
    bias_full = jnp.broadcast_to(b, (n_pad, d_out))

    sc = _make_sc_scatter(n_pad, h, ept)
    out = sc(sup0, sup1, rowp, colp, ewp, bias_full)
    return out[:n]
